# baseline (speedup 1.0000x reference)
import jax
import jax.numpy as jnp
from jax import lax
from jax.experimental import pallas as pl
from jax.experimental.pallas import tpu as pltpu
from jax.experimental.pallas import tpu_sc as plsc

D = 64
T = 200
B = 4096
NC, NS = 2, 16
NW = NC * NS

ROWS = B * T
ROWS_PER_W = ROWS // NW
IDX_W = 100
CHUNK_IR = 4
CHUNK_ROWS = CHUNK_IR * IDX_W
IR_PER_W = ROWS_PER_W // IDX_W
NCHUNK = IR_PER_W // CHUNK_IR
LANES = 16


def _emb_body(x_hbm, tok_hbm, pos_hbm, out_hbm, idx_v, rows_v, pos_v, sem):
    wid = lax.axis_index("s") * NC + lax.axis_index("c")
    pltpu.sync_copy(pos_hbm, pos_v)
    ir_base0 = wid * IR_PER_W

    def chunk_body(c, carry):
        ir_base = ir_base0 + c * CHUNK_IR
        pltpu.sync_copy(x_hbm.at[pl.ds(ir_base, CHUNK_IR)], idx_v)
        copies = [
            pltpu.async_copy(
                tok_hbm.at[idx_v.at[j]],
                rows_v.at[pl.ds(j * IDX_W, IDX_W)],
                sem,
            )
            for j in range(CHUNK_IR)
        ]
        for cp in copies:
            cp.wait()

        def add_body(r, carry2):
            for j in range(D // LANES):
                s = pl.ds(j * LANES, LANES)
                rows_v[r, s] = rows_v[r, s] + pos_v[r, s]
            return carry2

        lax.fori_loop(0, CHUNK_ROWS, add_body, 0, unroll=False)

        row_base = ir_base * IDX_W
        pltpu.sync_copy(rows_v, out_hbm.at[pl.ds(row_base, CHUNK_ROWS)])
        return carry

    lax.fori_loop(0, NCHUNK, chunk_body, 0, unroll=False)


@jax.jit
def kernel(x, token_table, pos_table):
    x2 = x.reshape(ROWS // IDX_W, IDX_W).astype(jnp.int32)
    pos2 = jnp.concatenate([pos_table, pos_table], axis=0)
    mesh = plsc.VectorSubcoreMesh(core_axis_name="c", subcore_axis_name="s")
    run = pl.kernel(
        _emb_body,
        mesh=mesh,
        compiler_params=pltpu.CompilerParams(use_tc_tiling_on_sc=False),
        out_type=jax.ShapeDtypeStruct((ROWS, D), jnp.float32),
        scratch_types=[
            pltpu.VMEM((CHUNK_IR, IDX_W), jnp.int32),
            pltpu.VMEM((CHUNK_ROWS, D), jnp.float32),
            pltpu.VMEM((2 * T, D), jnp.float32),
            pltpu.SemaphoreType.DMA,
        ],
    )
    out = run(x2, token_table, pos2)
    return out.reshape(B, T, D)


# speedup vs baseline: 1.0163x; 1.0163x over previous
import jax
import jax.numpy as jnp
from jax import lax
from jax.experimental import pallas as pl
from jax.experimental.pallas import tpu as pltpu
from jax.experimental.pallas import tpu_sc as plsc

D = 64
T = 200
B = 4096
NC, NS = 2, 16
NW = NC * NS
LANES = 16

ROWS = B * T
ROWS_PER_W = ROWS // NW
CHUNK = 128
NCHUNK = ROWS_PER_W // CHUNK
POS_ROWS = 336


def _emb_body(xw_hbm, tok_hbm, pos_hbm, out_hbm,
              idxw_v, wide_v, stage_v, pos_v, gsem, osem):
    wid = lax.axis_index("s") * NC + lax.axis_index("c")
    pltpu.sync_copy(pos_hbm, pos_v)
    row0 = wid * ROWS_PER_W
    ir0 = row0 // CHUNK

    def fetch(c, buf):
        pltpu.sync_copy(xw_hbm.at[ir0 + c], idxw_v.at[buf])
        pltpu.async_copy(tok_hbm.at[idxw_v.at[buf]], wide_v.at[buf], gsem)

    fetch(0, 0)

    def chunk_body(c, carry):
        buf = lax.rem(c, 2)
        pltpu.make_async_copy(tok_hbm.at[idxw_v.at[buf]], wide_v.at[buf],
                              gsem).wait()

        @pl.when(c + 1 < NCHUNK)
        def _():
            fetch(c + 1, 1 - buf)

        phase = lax.rem(c * CHUNK, T)

        @pl.when(c >= 2)
        def _():
            pltpu.make_async_copy(
                stage_v.at[buf],
                out_hbm.at[pl.ds(row0 + (c - 2) * CHUNK, CHUNK)],
                osem,
            ).wait()

        def row_body(r, carry2):
            for j in range(D // LANES):
                s = pl.ds(j * LANES, LANES)
                stage_v[buf, r, s] = wide_v[buf, r, s] + pos_v[phase + r, s]
            return carry2

        lax.fori_loop(0, CHUNK, row_body, 0, unroll=False)

        pltpu.async_copy(
            stage_v.at[buf],
            out_hbm.at[pl.ds(row0 + c * CHUNK, CHUNK)],
            osem,
        )
        return carry

    lax.fori_loop(0, NCHUNK, chunk_body, 0, unroll=False)
    for c in (NCHUNK - 2, NCHUNK - 1):
        pltpu.make_async_copy(
            stage_v.at[c % 2],
            out_hbm.at[pl.ds(row0 + c * CHUNK, CHUNK)],
            osem,
        ).wait()


@jax.jit
def kernel(x, token_table, pos_table):
    xw = x.astype(jnp.int32).reshape(ROWS // CHUNK, CHUNK)
    tok2 = jnp.pad(token_table, ((0, 0), (0, 128 - D)))
    rr = jnp.arange(POS_ROWS) % T
    pos2 = jnp.pad(pos_table[rr], ((0, 0), (0, 128 - D)))
    mesh = plsc.VectorSubcoreMesh(core_axis_name="c", subcore_axis_name="s")
    run = pl.kernel(
        _emb_body,
        mesh=mesh,
        out_type=jax.ShapeDtypeStruct((ROWS, D), jnp.float32),
        scratch_types=[
            pltpu.VMEM((2, CHUNK), jnp.int32),
            pltpu.VMEM((2, CHUNK, 128), jnp.float32),
            pltpu.VMEM((2, CHUNK, D), jnp.float32),
            pltpu.VMEM((POS_ROWS, 128), jnp.float32),
            pltpu.SemaphoreType.DMA,
            pltpu.SemaphoreType.DMA,
        ],
    )
    out = run(xw, tok2, pos2)
    return out.reshape(B, T, D)
